# baseline (speedup 1.0000x reference)
import functools

import jax
import jax.numpy as jnp
from jax import lax
from jax.experimental import pallas as pl
from jax.experimental.pallas import tpu as pltpu
from jax.experimental.pallas import tpu_sc as plsc

N_NODES = 10000
N_EDGES = 160000
D_FEAT = 256
D_PACK = D_FEAT // 2

NC = 2
NS = 16
NW = NC * NS
LANES = 16
CHUNK = 128
NCHUNK = 40
PER_W = CHUNK * NCHUNK
E_PAD = PER_W * NW

_BITREV = [int("{:04b}".format(i)[::-1], 2) for i in range(LANES)]
_HI_MASK = -65536


def _rot(v, s, idx):
    dnums = lax.GatherDimensionNumbers(
        offset_dims=(), collapsed_slice_dims=(0,), start_index_map=(0,))
    return lax.gather(v, (idx ^ s)[:, None], dnums, (1,),
                      mode=lax.GatherScatterMode.PROMISE_IN_BOUNDS)


def _tree16(vs, idx):
    level = vs
    for s in (8, 4, 2, 1):
        nxt = []
        for i in range(0, len(level), 2):
            a, b = level[i], level[i + 1]
            nxt.append(jnp.where((idx & (2 * s - 1)) < s,
                                 a + _rot(a, s, idx), b + _rot(b, s, idx)))
        level = nxt
    return level[0]


def _make_sc_kernel():
    mesh = plsc.VectorSubcoreMesh(core_axis_name="c", subcore_axis_name="s")

    @functools.partial(
        pl.kernel,
        mesh=mesh,
        out_type=jax.ShapeDtypeStruct((E_PAD,), jnp.float32),
        scratch_types=[
            pltpu.VMEM((PER_W,), jnp.int32),
            pltpu.VMEM((PER_W,), jnp.int32),
            pltpu.VMEM((CHUNK, D_PACK), jnp.int32),
            pltpu.VMEM((CHUNK, D_PACK), jnp.int32),
            pltpu.VMEM((CHUNK, D_PACK), jnp.int32),
            pltpu.VMEM((CHUNK, D_PACK), jnp.int32),
            pltpu.VMEM((PER_W,), jnp.float32),
            pltpu.SemaphoreType.DMA,
            pltpu.SemaphoreType.DMA,
            pltpu.SemaphoreType.DMA,
            pltpu.SemaphoreType.DMA,
        ],
    )
    def k(new_hbm, raw_hbm, src_hbm, dst_hbm, out_hbm,
          src_v, dst_v, u0, u1, v0, v1, out_all,
          sem_u0, sem_u1, sem_v0, sem_v1):
        ubuf = (u0, u1)
        vbuf = (v0, v1)
        usem = (sem_u0, sem_u1)
        vsem = (sem_v0, sem_v1)

        wid = lax.axis_index("s") * NC + lax.axis_index("c")
        base = wid * PER_W
        pltpu.sync_copy(src_hbm.at[pl.ds(base, PER_W)], src_v)
        pltpu.sync_copy(dst_hbm.at[pl.ds(base, PER_W)], dst_v)

        idx = lax.iota(jnp.int32, LANES)

        def start(c, b):
            off = c * CHUNK
            pltpu.async_copy(
                new_hbm.at[src_v.at[pl.ds(off, CHUNK)]], ubuf[b], usem[b])
            pltpu.async_copy(
                raw_hbm.at[dst_v.at[pl.ds(off, CHUNK)]], vbuf[b], vsem[b])

        start(0, 0)
        start(1, 1)

        def edge_acc(u_rows, v_rows, r):
            acc = None
            for w in range(D_PACK // LANES):
                uw = u_rows[r, pl.ds(w * LANES, LANES)]
                vw = v_rows[r, pl.ds(w * LANES, LANES)]
                u_lo = lax.bitcast_convert_type(uw << 16, jnp.float32)
                u_hi = lax.bitcast_convert_type(uw & _HI_MASK, jnp.float32)
                v_lo = lax.bitcast_convert_type(vw << 16, jnp.float32)
                v_hi = lax.bitcast_convert_type(vw & _HI_MASK, jnp.float32)
                term = u_lo * v_lo + u_hi * v_hi
                acc = term if acc is None else acc + term
            return acc

        def chunk_pair(jj, _):
            for b in range(2):
                c = 2 * jj + b
                pltpu.make_async_copy(
                    new_hbm.at[pl.ds(0, CHUNK)], ubuf[b], usem[b]).wait()
                pltpu.make_async_copy(
                    raw_hbm.at[pl.ds(0, CHUNK)], vbuf[b], vsem[b]).wait()

                def group_body(g, _, b=b, c=c):
                    accs = []
                    for p in range(LANES):
                        r = g * LANES + _BITREV[p]
                        accs.append(edge_acc(ubuf[b], vbuf[b], r))
                    out_all[pl.ds(c * CHUNK + g * LANES, LANES)] = (
                        _tree16(accs, idx))
                    return 0

                lax.fori_loop(0, CHUNK // LANES, group_body, 0)

                @pl.when(c + 2 < NCHUNK)
                def _(b=b, c=c):
                    start(c + 2, b)
            return 0

        lax.fori_loop(0, NCHUNK // 2, chunk_pair, 0)
        pltpu.sync_copy(out_all, out_hbm.at[pl.ds(base, PER_W)])

    return k


_sc_kernel = _make_sc_kernel()


@jax.jit
def kernel(new_ft, raw_ft, edge_index):
    new_p = lax.bitcast_convert_type(
        new_ft.astype(jnp.bfloat16).reshape(N_NODES, D_PACK, 2), jnp.int32)
    raw_p = lax.bitcast_convert_type(
        raw_ft.astype(jnp.bfloat16).reshape(N_NODES, D_PACK, 2), jnp.int32)
    src = edge_index[0].astype(jnp.int32)
    dst = edge_index[1].astype(jnp.int32)
    pad = jnp.zeros((E_PAD - N_EDGES,), jnp.int32)
    src = jnp.concatenate([src, pad])
    dst = jnp.concatenate([dst, pad])
    score = _sc_kernel(new_p, raw_p, src, dst)
    return score[:N_EDGES].reshape(N_EDGES, 1)


# speedup vs baseline: 1.3832x; 1.3832x over previous
import functools

import jax
import jax.numpy as jnp
from jax import lax
from jax.experimental import pallas as pl
from jax.experimental.pallas import tpu as pltpu
from jax.experimental.pallas import tpu_sc as plsc

N_NODES = 10000
N_EDGES = 160000
D_FEAT = 256

NC = 2
NS = 16
NW = NC * NS
LANES = 16
CHUNK = 96
NCHUNK = 53
PER_W = CHUNK * NCHUNK
E_PAD = PER_W * NW

_BITREV = [int("{:04b}".format(i)[::-1], 2) for i in range(LANES)]


def _rot(v, s, idx):
    dnums = lax.GatherDimensionNumbers(
        offset_dims=(), collapsed_slice_dims=(0,), start_index_map=(0,))
    return lax.gather(v, (idx ^ s)[:, None], dnums, (1,),
                      mode=lax.GatherScatterMode.PROMISE_IN_BOUNDS)


def _tree16(vs, idx):
    level = vs
    for s in (8, 4, 2, 1):
        nxt = []
        for i in range(0, len(level), 2):
            a, b = level[i], level[i + 1]
            nxt.append(jnp.where((idx & (2 * s - 1)) < s,
                                 a + _rot(a, s, idx), b + _rot(b, s, idx)))
        level = nxt
    return level[0]


def _make_sc_kernel():
    mesh = plsc.VectorSubcoreMesh(core_axis_name="c", subcore_axis_name="s")

    @functools.partial(
        pl.kernel,
        mesh=mesh,
        out_type=jax.ShapeDtypeStruct((E_PAD,), jnp.float32),
        scratch_types=[
            pltpu.VMEM((PER_W,), jnp.int32),
            pltpu.VMEM((PER_W,), jnp.int32),
            pltpu.VMEM((CHUNK, D_FEAT), jnp.float32),
            pltpu.VMEM((CHUNK, D_FEAT), jnp.float32),
            pltpu.VMEM((CHUNK, D_FEAT), jnp.float32),
            pltpu.VMEM((CHUNK, D_FEAT), jnp.float32),
            pltpu.VMEM((PER_W,), jnp.float32),
            pltpu.SemaphoreType.DMA,
            pltpu.SemaphoreType.DMA,
            pltpu.SemaphoreType.DMA,
            pltpu.SemaphoreType.DMA,
        ],
    )
    def k(new_hbm, raw_hbm, src_hbm, dst_hbm, out_hbm,
          src_v, dst_v, u0, u1, v0, v1, out_all,
          sem_u0, sem_u1, sem_v0, sem_v1):
        ubuf = (u0, u1)
        vbuf = (v0, v1)
        usem = (sem_u0, sem_u1)
        vsem = (sem_v0, sem_v1)

        wid = lax.axis_index("s") * NC + lax.axis_index("c")
        base = wid * PER_W
        pltpu.sync_copy(src_hbm.at[pl.ds(base, PER_W)], src_v)
        pltpu.sync_copy(dst_hbm.at[pl.ds(base, PER_W)], dst_v)

        idx = lax.iota(jnp.int32, LANES)

        def start(c, b):
            off = c * CHUNK
            pltpu.async_copy(
                new_hbm.at[src_v.at[pl.ds(off, CHUNK)]], ubuf[b], usem[b])
            pltpu.async_copy(
                raw_hbm.at[dst_v.at[pl.ds(off, CHUNK)]], vbuf[b], vsem[b])

        start(0, 0)
        start(1, 1)

        def wait(b):
            pltpu.make_async_copy(
                new_hbm.at[pl.ds(0, CHUNK)], ubuf[b], usem[b]).wait()
            pltpu.make_async_copy(
                raw_hbm.at[pl.ds(0, CHUNK)], vbuf[b], vsem[b]).wait()

        def edge_acc(u_rows, v_rows, r):
            acc = None
            for w in range(D_FEAT // LANES):
                uw = u_rows[r, pl.ds(w * LANES, LANES)]
                vw = v_rows[r, pl.ds(w * LANES, LANES)]
                term = uw * vw
                acc = term if acc is None else acc + term
            return acc

        def compute(c, b):
            def group_body(g, _):
                accs = []
                for p in range(LANES):
                    r = g * LANES + _BITREV[p]
                    accs.append(edge_acc(ubuf[b], vbuf[b], r))
                out_all[pl.ds(c * CHUNK + g * LANES, LANES)] = (
                    _tree16(accs, idx))
                return 0

            lax.fori_loop(0, CHUNK // LANES, group_body, 0)

        def chunk_pair(jj, _):
            for b in range(2):
                c = 2 * jj + b
                wait(b)
                compute(c, b)

                @pl.when(c + 2 < NCHUNK)
                def _(b=b, c=c):
                    start(c + 2, b)
            return 0

        lax.fori_loop(0, NCHUNK // 2, chunk_pair, 0)

        c_last = NCHUNK - 1
        b_last = c_last % 2
        wait(b_last)
        compute(c_last, b_last)
        pltpu.sync_copy(out_all, out_hbm.at[pl.ds(base, PER_W)])

    return k


_sc_kernel = _make_sc_kernel()


@jax.jit
def kernel(new_ft, raw_ft, edge_index):
    src = edge_index[0].astype(jnp.int32)
    dst = edge_index[1].astype(jnp.int32)
    pad = jnp.zeros((E_PAD - N_EDGES,), jnp.int32)
    src = jnp.concatenate([src, pad])
    dst = jnp.concatenate([dst, pad])
    score = _sc_kernel(new_ft, raw_ft, src, dst)
    return score[:N_EDGES].reshape(N_EDGES, 1)
